# Initial kernel scaffold; baseline (speedup 1.0000x reference)
import jax
import jax.numpy as jnp
from jax.experimental import pallas as pl


def kernel(x, W_enc, enc_bias, W_dec, dec_bias):
    raise NotImplementedError("write your pallas kernel here")


# speedup vs baseline: 18.0181x; 18.0181x over previous
import jax
import jax.numpy as jnp
from jax import lax
from jax.experimental import pallas as pl
from jax.experimental.pallas import tpu as pltpu

_D_MODEL = 1024
_D_SPARSE = 8192
_K = 64
_N_TOK = 4096

_TB_A = 128
_TB_B = 2048
_SB_B = 512


def _encode_select_body(x_ref, wet_ref, eb_ref, db_ref, hid_ref):
    xp = x_ref[...] - db_ref[...]
    pre = jnp.dot(xp, wet_ref[...], preferred_element_type=jnp.float32)
    pre = jnp.maximum(pre + eb_ref[...], 0.0)

    tb = pre.shape[0]

    def _count_ge(t_bits):
        t_f = lax.bitcast_convert_type(t_bits, jnp.float32)
        return jnp.sum((pre >= t_f).astype(jnp.float32), axis=1, keepdims=True)

    def _bisect_bits(it, carry):
        lo, hi = carry
        mid = lo + lax.shift_right_logical(hi - lo, 1)
        cnt = _count_ge(mid)
        take = cnt >= float(_K)
        lo = jnp.where(take, mid, lo)
        hi = jnp.where(take, hi, mid)
        return lo, hi

    lo0 = jnp.zeros((tb, 1), jnp.int32)
    hi0 = jnp.full((tb, 1), 0x7F800000, jnp.int32)
    lo, hi = lax.fori_loop(0, 31, _bisect_bits, (lo0, hi0))
    t_f = lax.bitcast_convert_type(lo, jnp.float32)
    n_ge = jnp.sum((pre >= t_f).astype(jnp.float32), axis=1, keepdims=True)
    n_gt = jnp.sum((pre > t_f).astype(jnp.float32), axis=1, keepdims=True)
    budget = float(_K) - n_gt
    n_ties = n_ge - n_gt

    row_ok = jnp.logical_or(n_ties == budget, lo == 0)
    hid_ref[...] = jnp.where(pre >= t_f, pre, 0.0)

    @pl.when(jnp.logical_not(jnp.all(row_ok)))
    def _slow_tie_path():
        idx = lax.broadcasted_iota(jnp.int32, pre.shape, 1)
        tie = pre == t_f

        def _g(cut):
            m = jnp.logical_and(tie, idx <= cut)
            return jnp.sum(m.astype(jnp.float32), axis=1, keepdims=True)

        def _bisect_idx(it, carry):
            lo2, hi2 = carry
            mid = lo2 + lax.shift_right_logical(hi2 - lo2, 1)
            ok = _g(mid) >= budget
            hi2 = jnp.where(ok, mid, hi2)
            lo2 = jnp.where(ok, lo2, mid)
            return lo2, hi2

        lo2 = jnp.full((tb, 1), -1, jnp.int32)
        hi2 = jnp.full((tb, 1), _D_SPARSE - 1, jnp.int32)
        lo2, hi2 = lax.fori_loop(0, 13, _bisect_idx, (lo2, hi2))
        keep = jnp.logical_or(pre > t_f,
                              jnp.logical_and(tie, idx <= hi2))
        hid_ref[...] = jnp.where(keep, pre, 0.0)


def _decode_body(hid_ref, wdt_ref, x_ref, db_ref, out_ref, l2_ref, rec_ref):
    j = pl.program_id(1)
    nj = pl.num_programs(1)
    i = pl.program_id(0)
    ni = pl.num_programs(0)

    h16 = hid_ref[...].astype(jnp.bfloat16)
    part = jnp.dot(h16, wdt_ref[...], preferred_element_type=jnp.float32)

    @pl.when(j == 0)
    def _init():
        out_ref[...] = part

    @pl.when(j != 0)
    def _acc():
        out_ref[...] = out_ref[...] + part

    @pl.when(j == nj - 1)
    def _finish():
        sae = out_ref[...] + db_ref[...]
        out_ref[...] = sae
        e = sae - x_ref[...]
        partial = jnp.sum(e * e, axis=(0, 1), keepdims=True)

        @pl.when(i == 0)
        def _set():
            l2_ref[...] = partial

        @pl.when(i != 0)
        def _add():
            l2_ref[...] = l2_ref[...] + partial

        @pl.when(i == ni - 1)
        def _rec():
            rec_ref[...] = l2_ref[...] * (1.0 / float(_N_TOK * _D_MODEL))


def kernel(x, W_enc, enc_bias, W_dec, dec_bias):
    wet = W_enc.T
    wdt = W_dec.T.astype(jnp.bfloat16)
    eb = enc_bias.reshape(1, _D_SPARSE)
    db = dec_bias.reshape(1, _D_MODEL)

    hidden = pl.pallas_call(
        _encode_select_body,
        grid=(_N_TOK // _TB_A,),
        in_specs=[
            pl.BlockSpec((_TB_A, _D_MODEL), lambda i: (i, 0)),
            pl.BlockSpec((_D_MODEL, _D_SPARSE), lambda i: (0, 0)),
            pl.BlockSpec((1, _D_SPARSE), lambda i: (0, 0)),
            pl.BlockSpec((1, _D_MODEL), lambda i: (0, 0)),
        ],
        out_specs=pl.BlockSpec((_TB_A, _D_SPARSE), lambda i: (i, 0)),
        out_shape=jax.ShapeDtypeStruct((_N_TOK, _D_SPARSE), jnp.float32),
    )(x, wet, eb, db)

    sae, l2, rec = pl.pallas_call(
        _decode_body,
        grid=(_N_TOK // _TB_B, _D_SPARSE // _SB_B),
        in_specs=[
            pl.BlockSpec((_TB_B, _SB_B), lambda i, j: (i, j)),
            pl.BlockSpec((_SB_B, _D_MODEL), lambda i, j: (j, 0)),
            pl.BlockSpec((_TB_B, _D_MODEL), lambda i, j: (i, 0)),
            pl.BlockSpec((1, _D_MODEL), lambda i, j: (0, 0)),
        ],
        out_specs=[
            pl.BlockSpec((_TB_B, _D_MODEL), lambda i, j: (i, 0)),
            pl.BlockSpec((1, 1), lambda i, j: (0, 0)),
            pl.BlockSpec((1, 1), lambda i, j: (0, 0)),
        ],
        out_shape=[
            jax.ShapeDtypeStruct((_N_TOK, _D_MODEL), jnp.float32),
            jax.ShapeDtypeStruct((1, 1), jnp.float32),
            jax.ShapeDtypeStruct((1, 1), jnp.float32),
        ],
    )(hidden, wdt, x, db)

    return sae, hidden, l2[0, 0], rec[0, 0]


# speedup vs baseline: 19.0783x; 1.0588x over previous
import jax
import jax.numpy as jnp
from jax import lax
from jax.experimental import pallas as pl
from jax.experimental.pallas import tpu as pltpu

_D_MODEL = 1024
_D_SPARSE = 8192
_K = 64
_N_TOK = 4096

_TB_A = 128
_TB_B = 512


def _encode_select_body(x_ref, wet_ref, eb_ref, db_ref, hid_ref, hid16_ref):
    xp = x_ref[...] - db_ref[...]
    pre = jnp.dot(xp, wet_ref[...], preferred_element_type=jnp.float32)
    pre = jnp.maximum(pre + eb_ref[...], 0.0)

    tb = pre.shape[0]

    def _count_ge(t_bits):
        t_f = lax.bitcast_convert_type(t_bits, jnp.float32)
        return jnp.sum((pre >= t_f).astype(jnp.float32), axis=1, keepdims=True)

    def _bisect_bits(it, carry):
        lo, hi, cnt_lo, cnt_hi = carry
        mid = lo + lax.shift_right_logical(hi - lo, 1)
        cnt = _count_ge(mid)
        take = cnt >= float(_K)
        lo = jnp.where(take, mid, lo)
        hi = jnp.where(take, hi, mid)
        cnt_lo = jnp.where(take, cnt, cnt_lo)
        cnt_hi = jnp.where(take, cnt_hi, cnt)
        return lo, hi, cnt_lo, cnt_hi

    lo0 = jnp.zeros((tb, 1), jnp.int32)
    hi0 = jnp.full((tb, 1), 0x7F800000, jnp.int32)
    c_lo0 = jnp.full((tb, 1), float(_D_SPARSE), jnp.float32)
    c_hi0 = jnp.zeros((tb, 1), jnp.float32)
    lo, hi, n_ge, n_gt = lax.fori_loop(
        0, 31, _bisect_bits, (lo0, hi0, c_lo0, c_hi0))
    t_f = lax.bitcast_convert_type(lo, jnp.float32)
    budget = float(_K) - n_gt
    n_ties = n_ge - n_gt

    row_ok = jnp.logical_or(n_ties == budget, lo == 0)
    hid = jnp.where(pre >= t_f, pre, 0.0)
    hid_ref[...] = hid
    hid16_ref[...] = hid.astype(jnp.bfloat16)

    @pl.when(jnp.logical_not(jnp.all(row_ok)))
    def _slow_tie_path():
        idx = lax.broadcasted_iota(jnp.int32, pre.shape, 1)
        tie = pre == t_f

        def _g(cut):
            m = jnp.logical_and(tie, idx <= cut)
            return jnp.sum(m.astype(jnp.float32), axis=1, keepdims=True)

        def _bisect_idx(it, carry):
            lo2, hi2 = carry
            mid = lo2 + lax.shift_right_logical(hi2 - lo2, 1)
            ok = _g(mid) >= budget
            hi2 = jnp.where(ok, mid, hi2)
            lo2 = jnp.where(ok, lo2, mid)
            return lo2, hi2

        lo2 = jnp.full((tb, 1), -1, jnp.int32)
        hi2 = jnp.full((tb, 1), _D_SPARSE - 1, jnp.int32)
        lo2, hi2 = lax.fori_loop(0, 13, _bisect_idx, (lo2, hi2))
        keep = jnp.logical_or(pre > t_f,
                              jnp.logical_and(tie, idx <= hi2))
        hid2 = jnp.where(keep, pre, 0.0)
        hid_ref[...] = hid2
        hid16_ref[...] = hid2.astype(jnp.bfloat16)


def _decode_body(hid16_ref, wdt_ref, x_ref, db_ref, out_ref, l2_ref, rec_ref):
    i = pl.program_id(0)
    ni = pl.num_programs(0)

    acc = jnp.dot(hid16_ref[...], wdt_ref[...],
                  preferred_element_type=jnp.float32)
    sae = acc + db_ref[...]
    out_ref[...] = sae
    e = sae - x_ref[...]
    partial = jnp.sum(e * e, axis=(0, 1), keepdims=True)

    @pl.when(i == 0)
    def _set():
        l2_ref[...] = partial

    @pl.when(i != 0)
    def _add():
        l2_ref[...] = l2_ref[...] + partial

    @pl.when(i == ni - 1)
    def _rec():
        rec_ref[...] = l2_ref[...] * (1.0 / float(_N_TOK * _D_MODEL))


def kernel(x, W_enc, enc_bias, W_dec, dec_bias):
    wet = W_enc.T
    wdt = W_dec.T.astype(jnp.bfloat16)
    eb = enc_bias.reshape(1, _D_SPARSE)
    db = dec_bias.reshape(1, _D_MODEL)

    hidden, hidden16 = pl.pallas_call(
        _encode_select_body,
        grid=(_N_TOK // _TB_A,),
        in_specs=[
            pl.BlockSpec((_TB_A, _D_MODEL), lambda i: (i, 0)),
            pl.BlockSpec((_D_MODEL, _D_SPARSE), lambda i: (0, 0)),
            pl.BlockSpec((1, _D_SPARSE), lambda i: (0, 0)),
            pl.BlockSpec((1, _D_MODEL), lambda i: (0, 0)),
        ],
        out_specs=[
            pl.BlockSpec((_TB_A, _D_SPARSE), lambda i: (i, 0)),
            pl.BlockSpec((_TB_A, _D_SPARSE), lambda i: (i, 0)),
        ],
        out_shape=[
            jax.ShapeDtypeStruct((_N_TOK, _D_SPARSE), jnp.float32),
            jax.ShapeDtypeStruct((_N_TOK, _D_SPARSE), jnp.bfloat16),
        ],
    )(x, wet, eb, db)

    sae, l2, rec = pl.pallas_call(
        _decode_body,
        grid=(_N_TOK // _TB_B,),
        in_specs=[
            pl.BlockSpec((_TB_B, _D_SPARSE), lambda i: (i, 0)),
            pl.BlockSpec((_D_SPARSE, _D_MODEL), lambda i: (0, 0)),
            pl.BlockSpec((_TB_B, _D_MODEL), lambda i: (i, 0)),
            pl.BlockSpec((1, _D_MODEL), lambda i: (0, 0)),
        ],
        out_specs=[
            pl.BlockSpec((_TB_B, _D_MODEL), lambda i: (i, 0)),
            pl.BlockSpec((1, 1), lambda i: (0, 0)),
            pl.BlockSpec((1, 1), lambda i: (0, 0)),
        ],
        out_shape=[
            jax.ShapeDtypeStruct((_N_TOK, _D_MODEL), jnp.float32),
            jax.ShapeDtypeStruct((1, 1), jnp.float32),
            jax.ShapeDtypeStruct((1, 1), jnp.float32),
        ],
    )(hidden16, wdt, x, db)

    return sae, hidden, l2[0, 0], rec[0, 0]
